# Initial kernel scaffold; baseline (speedup 1.0000x reference)
import jax
import jax.numpy as jnp
from jax.experimental import pallas as pl


def kernel(x, edge_index, W0, b0, lin1, u1, c1, bias1, lin2, u2, c2, bias2, lin3, u3, c3, bias3, W1, bfc1, W2, bfc2):
    raise NotImplementedError("write your pallas kernel here")


# speedup vs baseline: 1.4667x; 1.4667x over previous
import functools
import math

import jax
import jax.numpy as jnp
from jax import lax
from jax.experimental import pallas as pl
from jax.experimental.pallas import tpu as pltpu
from jax.experimental.pallas import tpu_sc as plsc

N_NODES = 10000
N_PAD = 10240
N_EDGES = 320000
HEADS = 8
B_EDGE = 80
N_SC = 2
N_SUB = 16
ROWS_PER_SUB = N_PAD // N_SUB
EDGES_PER_SUB = N_EDGES // (N_SC * N_SUB)
BLOCKS_PER_SUB = EDGES_PER_SUB // B_EDGE


def _take(v, idx):
    dnums = lax.GatherDimensionNumbers(offset_dims=(), collapsed_slice_dims=(0,),
                                       start_index_map=(0,))
    return lax.gather(v, idx[:, None], dnums, (1,),
                      mode=lax.GatherScatterMode.PROMISE_IN_BOUNDS)


def _make_edge_kernel(in_c, num_groups, with_deg):
    h_grp = HEADS // num_groups
    assert h_grp * in_c == 128
    kpv = in_c // 16

    mesh = plsc.VectorSubcoreMesh(core_axis_name="c", subcore_axis_name="s")
    out_type = [jax.ShapeDtypeStruct((N_SC, num_groups, N_PAD, 128), jnp.float32)]
    if with_deg:
        out_type.append(jax.ShapeDtypeStruct((N_SC, N_PAD, 16), jnp.float32))
    scratch = [
        pltpu.VMEM((B_EDGE,), jnp.int32),
        pltpu.VMEM((B_EDGE,), jnp.int32),
        pltpu.VMEM((B_EDGE, in_c), jnp.float32),
        pltpu.VMEM((B_EDGE, 16), jnp.float32),
        pltpu.VMEM((B_EDGE, 16), jnp.float32),
        pltpu.VMEM((B_EDGE, 128), jnp.float32),
        pltpu.VMEM((16,), jnp.float32),
        pltpu.VMEM((B_EDGE, 16), jnp.float32),
        pltpu.VMEM_SHARED((N_PAD, 128), jnp.float32),
        pltpu.VMEM_SHARED((N_PAD, 16), jnp.float32),
        pltpu.SemaphoreType.DMA,
    ]

    @functools.partial(
        pl.kernel, mesh=mesh, out_type=out_type, scratch_types=scratch,
        compiler_params=pltpu.CompilerParams(use_tc_tiling_on_sc=False))
    def edge_kernel(src_hbm, dst_hbm, h_hbm, a_hbm, c_hbm, zeros_hbm, zeros16_hbm,
                    ones_hbm, *rest):
        if with_deg:
            s_out, deg_out = rest[0], rest[1]
            rest = rest[2:]
        else:
            s_out = rest[0]
            rest = rest[1:]
        (idx_s, idx_d, hrow, asrc, adst, msg, cv, onesv, s_sh, deg_sh,
         sem) = rest

        core = lax.axis_index("c")
        sub = lax.axis_index("s")
        ebase = core * (N_EDGES // N_SC) + sub * EDGES_PER_SUB
        rbase = sub * ROWS_PER_SUB

        pltpu.sync_copy(c_hbm, cv)
        if with_deg:
            pltpu.sync_copy(ones_hbm, onesv)

        lane = lax.iota(jnp.int32, 16)
        perms = [jnp.bitwise_xor(lane, jnp.int32(s)) for s in (4, 2, 1)]
        head_idx = [lane * 0 + jnp.int32(h) for h in range(HEADS)]

        for g in range(num_groups):
            pltpu.sync_copy(zeros_hbm.at[pl.ds(rbase, ROWS_PER_SUB)],
                            s_sh.at[pl.ds(rbase, ROWS_PER_SUB)])
            if with_deg and g == 0:
                pltpu.sync_copy(zeros16_hbm.at[pl.ds(rbase, ROWS_PER_SUB)],
                                deg_sh.at[pl.ds(rbase, ROWS_PER_SUB)])
            plsc.subcore_barrier()

            def block_body(i, carry):
                base = ebase + i * B_EDGE
                pltpu.sync_copy(src_hbm.at[pl.ds(base, B_EDGE)], idx_s)
                pltpu.sync_copy(dst_hbm.at[pl.ds(base, B_EDGE)], idx_d)
                pltpu.async_copy(h_hbm.at[idx_s], hrow, sem).wait()
                pltpu.async_copy(a_hbm.at[idx_s], asrc, sem).wait()
                pltpu.async_copy(a_hbm.at[idx_d], adst, sem).wait()
                cvec = cv[...]

                def edge_body(e, carry2):
                    d = asrc[e, :] - adst[e, :] + cvec
                    m = d
                    for p in perms:
                        m = jnp.maximum(m, _take(m, p))
                    ex = jnp.exp(d - m)
                    sm = ex
                    for p in perms:
                        sm = sm + _take(sm, p)
                    q = ex / sm
                    hk = [hrow[e, pl.ds(k * 16, 16)] for k in range(kpv)]
                    for h in range(h_grp):
                        qh = _take(q, head_idx[g * h_grp + h])
                        for k in range(kpv):
                            msg[e, pl.ds(h * in_c + k * 16, 16)] = qh * hk[k]
                    return carry2

                lax.fori_loop(0, B_EDGE, edge_body, 0)
                pltpu.sync_copy(msg, s_sh.at[idx_d], add=True)
                if with_deg and g == 0:
                    pltpu.sync_copy(onesv, deg_sh.at[idx_d], add=True)
                return carry

            lax.fori_loop(0, BLOCKS_PER_SUB, block_body, 0)
            plsc.subcore_barrier()
            pltpu.sync_copy(s_sh.at[pl.ds(rbase, ROWS_PER_SUB)],
                            s_out.at[core, g, pl.ds(rbase, ROWS_PER_SUB)])
            if with_deg and g == 0:
                pltpu.sync_copy(deg_sh.at[pl.ds(rbase, ROWS_PER_SUB)],
                                deg_out.at[core, pl.ds(rbase, ROWS_PER_SUB)])

    return edge_kernel


def _elu(v):
    return jnp.where(v > 0, v, jnp.exp(v) - 1.0)


_BM = 1024


def _agg_call(s_all, wt, bias, deg, u):
    parts, groups = s_all.shape[0], s_all.shape[1]
    oc = wt.shape[2]
    has_deg = deg is not None
    emit_a = u is not None

    def body(*refs):
        refs = list(refs)
        s_ref = refs.pop(0)
        wt_ref = refs.pop(0)
        b_ref = refs.pop(0)
        deg_ref = refs.pop(0) if has_deg else None
        u_ref = refs.pop(0) if emit_a else None
        h_ref = refs.pop(0)
        a_ref = refs.pop(0) if emit_a else None

        acc = jnp.zeros((_BM, oc), jnp.float32)
        for p in range(parts):
            for g in range(groups):
                acc = acc + jnp.dot(s_ref[p, g], wt_ref[g],
                                    preferred_element_type=jnp.float32,
                                    precision=lax.Precision.HIGHEST)
        if has_deg:
            dg = deg_ref[0, :, 0:1] + deg_ref[1, :, 0:1]
            acc = acc / jnp.clip(dg, 1.0, None)
        h = _elu(acc + b_ref[0])
        h_ref[...] = h
        if emit_a:
            a = jnp.dot(h, u_ref[...], preferred_element_type=jnp.float32,
                        precision=lax.Precision.HIGHEST)
            a_ref[...] = jnp.concatenate([a, a], axis=1)

    grid = N_PAD // _BM
    in_specs = [
        pl.BlockSpec((parts, groups, _BM, 128), lambda i: (0, 0, i, 0)),
        pl.BlockSpec((groups, 128, oc), lambda i: (0, 0, 0)),
        pl.BlockSpec((1, oc), lambda i: (0, 0)),
    ]
    args = [s_all, wt, bias.reshape(1, oc)]
    if has_deg:
        in_specs.append(pl.BlockSpec((2, _BM, 16), lambda i: (0, i, 0)))
        args.append(deg)
    if emit_a:
        in_specs.append(pl.BlockSpec((oc, HEADS), lambda i: (0, 0)))
        args.append(u)
    out_shape = [jax.ShapeDtypeStruct((N_PAD, oc), jnp.float32)]
    out_specs = [pl.BlockSpec((_BM, oc), lambda i: (i, 0))]
    if emit_a:
        out_shape.append(jax.ShapeDtypeStruct((N_PAD, 16), jnp.float32))
        out_specs.append(pl.BlockSpec((_BM, 16), lambda i: (i, 0)))
    res = pl.pallas_call(body, grid=(grid,), in_specs=in_specs,
                         out_specs=out_specs, out_shape=out_shape)(*args)
    return res if emit_a else (res[0], None)


def _final_call(h, w1, b1, w2p, b2p):
    def body(h_ref, w1_ref, b1_ref, w2_ref, b2_ref, o_ref):
        m = _elu(jnp.dot(h_ref[...], w1_ref[...],
                         preferred_element_type=jnp.float32,
                         precision=lax.Precision.HIGHEST) + b1_ref[0])
        o_ref[...] = jnp.dot(m, w2_ref[...],
                             preferred_element_type=jnp.float32,
                             precision=lax.Precision.HIGHEST) + b2_ref[0]

    grid = N_PAD // _BM
    return pl.pallas_call(
        body, grid=(grid,),
        in_specs=[
            pl.BlockSpec((_BM, 128), lambda i: (i, 0)),
            pl.BlockSpec((128, 256), lambda i: (0, 0)),
            pl.BlockSpec((1, 256), lambda i: (0, 0)),
            pl.BlockSpec((256, 16), lambda i: (0, 0)),
            pl.BlockSpec((1, 16), lambda i: (0, 0)),
        ],
        out_specs=pl.BlockSpec((_BM, 16), lambda i: (i, 0)),
        out_shape=jax.ShapeDtypeStruct((N_PAD, 16), jnp.float32),
    )(h, w1, b1.reshape(1, 256), w2p, b2p)


def _wt_layout(lin, in_c, oc):
    w = lin.reshape(in_c, HEADS, oc).transpose(1, 0, 2).reshape(HEADS * in_c, oc)
    return w.reshape(-1, 128, oc)


_EDGE_CACHE = {}


def _edge(in_c, num_groups, with_deg):
    key = (in_c, num_groups, with_deg)
    if key not in _EDGE_CACHE:
        _EDGE_CACHE[key] = _make_edge_kernel(in_c, num_groups, with_deg)
    return _EDGE_CACHE[key]


def kernel(x, edge_index, W0, b0, lin1, u1, c1, bias1, lin2, u2, c2, bias2,
           lin3, u3, c3, bias3, W1, bfc1, W2, bfc2):
    src = edge_index[0]
    dst = edge_index[1]
    xp = jnp.pad(x, ((0, N_PAD - N_NODES), (0, 0)))
    zeros = jnp.zeros((N_PAD, 128), jnp.float32)
    zeros16 = jnp.zeros((N_PAD, 16), jnp.float32)
    ones = jnp.ones((B_EDGE, 16), jnp.float32)
    c1t = jnp.concatenate([c1, c1])
    c2t = jnp.concatenate([c2, c2])
    c3t = jnp.concatenate([c3, c3])

    h0, a1 = _agg_call(xp.reshape(1, 1, N_PAD, 128), W0.reshape(1, 128, 16),
                       b0, None, u1)
    s1, deg = _edge(16, 1, True)(src, dst, h0, a1, c1t, zeros, zeros16, ones)
    h1, a2 = _agg_call(s1, _wt_layout(lin1, 16, 32), bias1, deg, u2)
    (s2,) = _edge(32, 2, False)(src, dst, h1, a2, c2t, zeros, zeros16, ones)
    h2, a3 = _agg_call(s2, _wt_layout(lin2, 32, 64), bias2, deg, u3)
    (s3,) = _edge(64, 4, False)(src, dst, h2, a3, c3t, zeros, zeros16, ones)
    h3, _ = _agg_call(s3, _wt_layout(lin3, 64, 128), bias3, deg, None)

    w2p = jnp.zeros((256, 16), jnp.float32).at[:, :10].set(W2)
    b2p = jnp.zeros((1, 16), jnp.float32).at[0, :10].set(bfc2)
    out = _final_call(h3, W1, bfc1, w2p, b2p)
    return out[:N_NODES, :10]


# speedup vs baseline: 2.2421x; 1.5287x over previous
import functools
import math

import jax
import jax.numpy as jnp
from jax import lax
from jax.experimental import pallas as pl
from jax.experimental.pallas import tpu as pltpu
from jax.experimental.pallas import tpu_sc as plsc

N_NODES = 10000
N_PAD = 10240
N_EDGES = 320000
HEADS = 8
B_EDGE = 80
N_SC = 2
N_SUB = 16
ROWS_PER_SUB = N_PAD // N_SUB
EDGES_PER_SUB = N_EDGES // (N_SC * N_SUB)
BLOCKS_PER_SUB = EDGES_PER_SUB // B_EDGE


def _take(v, idx):
    dnums = lax.GatherDimensionNumbers(offset_dims=(), collapsed_slice_dims=(0,),
                                       start_index_map=(0,))
    return lax.gather(v, idx[:, None], dnums, (1,),
                      mode=lax.GatherScatterMode.PROMISE_IN_BOUNDS)


def _make_edge_kernel(in_c, num_groups, with_deg):
    h_grp = HEADS // num_groups
    assert h_grp * in_c == 128
    kpv = in_c // 16
    write_q = num_groups > 1

    mesh = plsc.VectorSubcoreMesh(core_axis_name="c", subcore_axis_name="s")
    out_type = [jax.ShapeDtypeStruct((N_SC, num_groups, N_PAD, 128), jnp.float32)]
    if with_deg:
        out_type.append(jax.ShapeDtypeStruct((N_SC, N_PAD, 16), jnp.float32))
    if write_q:
        out_type.append(jax.ShapeDtypeStruct((N_EDGES, 16), jnp.float32))
    scratch = [
        pltpu.VMEM((B_EDGE,), jnp.int32),
        pltpu.VMEM((B_EDGE,), jnp.int32),
        pltpu.VMEM((B_EDGE, in_c), jnp.float32),
        pltpu.VMEM((B_EDGE, 16), jnp.float32),
        pltpu.VMEM((B_EDGE, 16), jnp.float32),
        pltpu.VMEM((B_EDGE, 16), jnp.float32),
        pltpu.VMEM((B_EDGE, 128), jnp.float32),
        pltpu.VMEM((16,), jnp.float32),
        pltpu.VMEM((B_EDGE, 16), jnp.float32),
        pltpu.VMEM_SHARED((N_PAD, 128), jnp.float32),
        pltpu.VMEM_SHARED((N_PAD, 16), jnp.float32),
        pltpu.SemaphoreType.DMA,
    ]

    @functools.partial(
        pl.kernel, mesh=mesh, out_type=out_type, scratch_types=scratch,
        compiler_params=pltpu.CompilerParams(use_tc_tiling_on_sc=False))
    def edge_kernel(src_hbm, dst_hbm, h_hbm, a_hbm, c_hbm, zeros_hbm, zeros16_hbm,
                    ones_hbm, *rest):
        rest = list(rest)
        s_out = rest.pop(0)
        deg_out = rest.pop(0) if with_deg else None
        q_out = rest.pop(0) if write_q else None
        (idx_s, idx_d, hrow, asrc, adst, qblk, msg, cv, onesv, s_sh, deg_sh,
         sem) = rest

        core = lax.axis_index("c")
        sub = lax.axis_index("s")
        ebase = core * (N_EDGES // N_SC) + sub * EDGES_PER_SUB
        rbase = sub * ROWS_PER_SUB

        pltpu.sync_copy(c_hbm, cv)
        if with_deg:
            pltpu.sync_copy(ones_hbm, onesv)

        lane = lax.iota(jnp.int32, 16)
        perms = [jnp.bitwise_xor(lane, jnp.int32(s)) for s in (4, 2, 1)]
        head_idx = [lane * 0 + jnp.int32(h) for h in range(HEADS)]

        for g in range(num_groups):
            pltpu.sync_copy(zeros_hbm.at[pl.ds(rbase, ROWS_PER_SUB)],
                            s_sh.at[pl.ds(rbase, ROWS_PER_SUB)])
            if with_deg and g == 0:
                pltpu.sync_copy(zeros16_hbm.at[pl.ds(rbase, ROWS_PER_SUB)],
                                deg_sh.at[pl.ds(rbase, ROWS_PER_SUB)])
            plsc.subcore_barrier()

            def block_body(i, carry):
                base = ebase + i * B_EDGE
                pltpu.sync_copy(dst_hbm.at[pl.ds(base, B_EDGE)], idx_d)
                pltpu.sync_copy(src_hbm.at[pl.ds(base, B_EDGE)], idx_s)
                pltpu.async_copy(h_hbm.at[idx_s], hrow, sem).wait()
                if g == 0:
                    pltpu.async_copy(a_hbm.at[idx_s], asrc, sem).wait()
                    pltpu.async_copy(a_hbm.at[idx_d], adst, sem).wait()
                    cvec = cv[...]
                    for e in range(B_EDGE):
                        d = asrc[e, :] - adst[e, :] + cvec
                        m = d
                        for p in perms:
                            m = jnp.maximum(m, _take(m, p))
                        ex = jnp.exp(d - m)
                        sm = ex
                        for p in perms:
                            sm = sm + _take(sm, p)
                        q = ex / sm
                        if write_q:
                            qblk[e, :] = q
                        for h in range(h_grp):
                            qh = _take(q, head_idx[h])
                            for k in range(kpv):
                                msg[e, pl.ds(h * in_c + k * 16, 16)] = (
                                    qh * hrow[e, pl.ds(k * 16, 16)])
                    if write_q:
                        pltpu.sync_copy(qblk, q_out.at[pl.ds(base, B_EDGE)])
                else:
                    pltpu.sync_copy(q_out.at[pl.ds(base, B_EDGE)], qblk)
                    for e in range(B_EDGE):
                        q = qblk[e, :]
                        for h in range(h_grp):
                            qh = _take(q, head_idx[g * h_grp + h])
                            for k in range(kpv):
                                msg[e, pl.ds(h * in_c + k * 16, 16)] = (
                                    qh * hrow[e, pl.ds(k * 16, 16)])
                pltpu.sync_copy(msg, s_sh.at[idx_d], add=True)
                if with_deg and g == 0:
                    pltpu.sync_copy(onesv, deg_sh.at[idx_d], add=True)
                return carry

            lax.fori_loop(0, BLOCKS_PER_SUB, block_body, 0)
            plsc.subcore_barrier()
            pltpu.sync_copy(s_sh.at[pl.ds(rbase, ROWS_PER_SUB)],
                            s_out.at[core, g, pl.ds(rbase, ROWS_PER_SUB)])
            if with_deg and g == 0:
                pltpu.sync_copy(deg_sh.at[pl.ds(rbase, ROWS_PER_SUB)],
                                deg_out.at[core, pl.ds(rbase, ROWS_PER_SUB)])

    return edge_kernel


def _elu(v):
    return jnp.where(v > 0, v, jnp.exp(v) - 1.0)


_BM = 1024


def _agg_call(s_all, wt, bias, deg, u):
    parts, groups = s_all.shape[0], s_all.shape[1]
    oc = wt.shape[2]
    has_deg = deg is not None
    emit_a = u is not None

    def body(*refs):
        refs = list(refs)
        s_ref = refs.pop(0)
        wt_ref = refs.pop(0)
        b_ref = refs.pop(0)
        deg_ref = refs.pop(0) if has_deg else None
        u_ref = refs.pop(0) if emit_a else None
        h_ref = refs.pop(0)
        a_ref = refs.pop(0) if emit_a else None

        acc = jnp.zeros((_BM, oc), jnp.float32)
        for p in range(parts):
            for g in range(groups):
                acc = acc + jnp.dot(s_ref[p, g], wt_ref[g],
                                    preferred_element_type=jnp.float32,
                                    precision=lax.Precision.HIGHEST)
        if has_deg:
            dg = deg_ref[0, :, 0:1] + deg_ref[1, :, 0:1]
            acc = acc / jnp.clip(dg, 1.0, None)
        h = _elu(acc + b_ref[0])
        h_ref[...] = h
        if emit_a:
            a = jnp.dot(h, u_ref[...], preferred_element_type=jnp.float32,
                        precision=lax.Precision.HIGHEST)
            a_ref[...] = jnp.concatenate([a, a], axis=1)

    grid = N_PAD // _BM
    in_specs = [
        pl.BlockSpec((parts, groups, _BM, 128), lambda i: (0, 0, i, 0)),
        pl.BlockSpec((groups, 128, oc), lambda i: (0, 0, 0)),
        pl.BlockSpec((1, oc), lambda i: (0, 0)),
    ]
    args = [s_all, wt, bias.reshape(1, oc)]
    if has_deg:
        in_specs.append(pl.BlockSpec((2, _BM, 16), lambda i: (0, i, 0)))
        args.append(deg)
    if emit_a:
        in_specs.append(pl.BlockSpec((oc, HEADS), lambda i: (0, 0)))
        args.append(u)
    out_shape = [jax.ShapeDtypeStruct((N_PAD, oc), jnp.float32)]
    out_specs = [pl.BlockSpec((_BM, oc), lambda i: (i, 0))]
    if emit_a:
        out_shape.append(jax.ShapeDtypeStruct((N_PAD, 16), jnp.float32))
        out_specs.append(pl.BlockSpec((_BM, 16), lambda i: (i, 0)))
    res = pl.pallas_call(body, grid=(grid,), in_specs=in_specs,
                         out_specs=out_specs, out_shape=out_shape)(*args)
    return res if emit_a else (res[0], None)


def _final_call(h, w1, b1, w2p, b2p):
    def body(h_ref, w1_ref, b1_ref, w2_ref, b2_ref, o_ref):
        m = _elu(jnp.dot(h_ref[...], w1_ref[...],
                         preferred_element_type=jnp.float32,
                         precision=lax.Precision.HIGHEST) + b1_ref[0])
        o_ref[...] = jnp.dot(m, w2_ref[...],
                             preferred_element_type=jnp.float32,
                             precision=lax.Precision.HIGHEST) + b2_ref[0]

    grid = N_PAD // _BM
    return pl.pallas_call(
        body, grid=(grid,),
        in_specs=[
            pl.BlockSpec((_BM, 128), lambda i: (i, 0)),
            pl.BlockSpec((128, 256), lambda i: (0, 0)),
            pl.BlockSpec((1, 256), lambda i: (0, 0)),
            pl.BlockSpec((256, 16), lambda i: (0, 0)),
            pl.BlockSpec((1, 16), lambda i: (0, 0)),
        ],
        out_specs=pl.BlockSpec((_BM, 16), lambda i: (i, 0)),
        out_shape=jax.ShapeDtypeStruct((N_PAD, 16), jnp.float32),
    )(h, w1, b1.reshape(1, 256), w2p, b2p)


def _wt_layout(lin, in_c, oc):
    w = lin.reshape(in_c, HEADS, oc).transpose(1, 0, 2).reshape(HEADS * in_c, oc)
    return w.reshape(-1, 128, oc)


_EDGE_CACHE = {}


def _edge(in_c, num_groups, with_deg):
    key = (in_c, num_groups, with_deg)
    if key not in _EDGE_CACHE:
        _EDGE_CACHE[key] = _make_edge_kernel(in_c, num_groups, with_deg)
    return _EDGE_CACHE[key]


def kernel(x, edge_index, W0, b0, lin1, u1, c1, bias1, lin2, u2, c2, bias2,
           lin3, u3, c3, bias3, W1, bfc1, W2, bfc2):
    src = edge_index[0]
    dst = edge_index[1]
    xp = jnp.pad(x, ((0, N_PAD - N_NODES), (0, 0)))
    zeros = jnp.zeros((N_PAD, 128), jnp.float32)
    zeros16 = jnp.zeros((N_PAD, 16), jnp.float32)
    ones = jnp.ones((B_EDGE, 16), jnp.float32)
    c1t = jnp.concatenate([c1, c1])
    c2t = jnp.concatenate([c2, c2])
    c3t = jnp.concatenate([c3, c3])

    h0, a1 = _agg_call(xp.reshape(1, 1, N_PAD, 128), W0.reshape(1, 128, 16),
                       b0, None, u1)
    s1, deg = _edge(16, 1, True)(src, dst, h0, a1, c1t, zeros, zeros16, ones)
    h1, a2 = _agg_call(s1, _wt_layout(lin1, 16, 32), bias1, deg, u2)
    s2, _ = _edge(32, 2, False)(src, dst, h1, a2, c2t, zeros, zeros16, ones)
    h2, a3 = _agg_call(s2, _wt_layout(lin2, 32, 64), bias2, deg, u3)
    s3, _ = _edge(64, 4, False)(src, dst, h2, a3, c3t, zeros, zeros16, ones)
    h3, _ = _agg_call(s3, _wt_layout(lin3, 64, 128), bias3, deg, None)

    w2p = jnp.zeros((256, 16), jnp.float32).at[:, :10].set(W2)
    b2p = jnp.zeros((1, 16), jnp.float32).at[0, :10].set(bfc2)
    out = _final_call(h3, W1, bfc1, w2p, b2p)
    return out[:N_NODES, :10]


# speedup vs baseline: 2.7437x; 1.2237x over previous
import functools
import math

import jax
import jax.numpy as jnp
from jax import lax
from jax.experimental import pallas as pl
from jax.experimental.pallas import tpu as pltpu
from jax.experimental.pallas import tpu_sc as plsc

N_NODES = 10000
N_PAD = 10240
N_EDGES = 320000
HEADS = 8
B_EDGE = 80
N_SC = 2
N_SUB = 16
ROWS_PER_SUB = N_PAD // N_SUB
EDGES_PER_SUB = N_EDGES // (N_SC * N_SUB)
BLOCKS_PER_SUB = EDGES_PER_SUB // B_EDGE


def _take(v, idx):
    dnums = lax.GatherDimensionNumbers(offset_dims=(), collapsed_slice_dims=(0,),
                                       start_index_map=(0,))
    return lax.gather(v, idx[:, None], dnums, (1,),
                      mode=lax.GatherScatterMode.PROMISE_IN_BOUNDS)


def _make_edge_kernel(in_c, num_groups, with_deg):
    h_grp = HEADS // num_groups
    assert h_grp * in_c == 128
    kpv = in_c // 16
    write_q = num_groups > 1

    mesh = plsc.VectorSubcoreMesh(core_axis_name="c", subcore_axis_name="s")
    out_type = [jax.ShapeDtypeStruct((N_SC, num_groups, N_PAD, 128), jnp.float32)]
    if with_deg:
        out_type.append(jax.ShapeDtypeStruct((N_SC, N_PAD, 16), jnp.float32))
    if write_q:
        out_type.append(jax.ShapeDtypeStruct((N_EDGES, 16), jnp.float32))
    scratch = [
        pltpu.VMEM((2, B_EDGE), jnp.int32),
        pltpu.VMEM((2, B_EDGE), jnp.int32),
        pltpu.VMEM((2, B_EDGE, in_c), jnp.float32),
        pltpu.VMEM((2, B_EDGE, 16), jnp.float32),
        pltpu.VMEM((2, B_EDGE, 16), jnp.float32),
        pltpu.VMEM((2, B_EDGE, 16), jnp.float32),
        pltpu.VMEM((B_EDGE, 128), jnp.float32),
        pltpu.VMEM((16,), jnp.float32),
        pltpu.VMEM((B_EDGE, 16), jnp.float32),
        pltpu.VMEM_SHARED((N_PAD, 128), jnp.float32),
        pltpu.VMEM_SHARED((N_PAD, 16), jnp.float32),
        pltpu.SemaphoreType.DMA,
        pltpu.SemaphoreType.DMA,
    ]

    @functools.partial(
        pl.kernel, mesh=mesh, out_type=out_type, scratch_types=scratch,
        compiler_params=pltpu.CompilerParams(use_tc_tiling_on_sc=False))
    def edge_kernel(src_hbm, dst_hbm, h_hbm, a_hbm, c_hbm, zeros_hbm, zeros16_hbm,
                    ones_hbm, *rest):
        rest = list(rest)
        s_out = rest.pop(0)
        deg_out = rest.pop(0) if with_deg else None
        q_out = rest.pop(0) if write_q else None
        (idx_s, idx_d, hrow, asrc, adst, qblk, msg, cv, onesv, s_sh, deg_sh,
         sem0, sem1) = rest
        sems = (sem0, sem1)

        core = lax.axis_index("c")
        sub = lax.axis_index("s")
        ebase = core * (N_EDGES // N_SC) + sub * EDGES_PER_SUB
        rbase = sub * ROWS_PER_SUB

        pltpu.sync_copy(c_hbm, cv)
        if with_deg:
            pltpu.sync_copy(ones_hbm, onesv)

        lane = lax.iota(jnp.int32, 16)
        perms = [jnp.bitwise_xor(lane, jnp.int32(s)) for s in (4, 2, 1)]
        head_idx = [lane * 0 + jnp.int32(h) for h in range(HEADS)]

        for g in range(num_groups):
            pltpu.sync_copy(zeros_hbm.at[pl.ds(rbase, ROWS_PER_SUB)],
                            s_sh.at[pl.ds(rbase, ROWS_PER_SUB)])
            if with_deg and g == 0:
                pltpu.sync_copy(zeros16_hbm.at[pl.ds(rbase, ROWS_PER_SUB)],
                                deg_sh.at[pl.ds(rbase, ROWS_PER_SUB)])
            plsc.subcore_barrier()

            def gather_descs(b):
                ds = [pltpu.make_async_copy(h_hbm.at[idx_s.at[b]],
                                            hrow.at[b], sems[b])]
                if g == 0:
                    ds.append(pltpu.make_async_copy(a_hbm.at[idx_s.at[b]],
                                                    asrc.at[b], sems[b]))
                    ds.append(pltpu.make_async_copy(a_hbm.at[idx_d.at[b]],
                                                    adst.at[b], sems[b]))
                return ds

            def prefetch(t, b):
                base = ebase + t * B_EDGE
                pltpu.sync_copy(dst_hbm.at[pl.ds(base, B_EDGE)], idx_d.at[b])
                pltpu.sync_copy(src_hbm.at[pl.ds(base, B_EDGE)], idx_s.at[b])
                pltpu.async_copy(h_hbm.at[idx_s.at[b]], hrow.at[b], sems[b])
                if g == 0:
                    pltpu.async_copy(a_hbm.at[idx_s.at[b]], asrc.at[b],
                                     sems[b])
                    pltpu.async_copy(a_hbm.at[idx_d.at[b]], adst.at[b],
                                     sems[b])
                else:
                    pltpu.async_copy(q_out.at[pl.ds(base, B_EDGE)],
                                     qblk.at[b], sems[b])

            def compute(i, b):
                base = ebase + jnp.where(i < BLOCKS_PER_SUB, i, 0) * B_EDGE
                for d_ in gather_descs(b):
                    d_.wait()
                if g == 0:
                    cvec = cv[...]
                    for e in range(B_EDGE):
                        d = asrc[b, e, :] - adst[b, e, :] + cvec
                        m = d
                        for p in perms:
                            m = jnp.maximum(m, _take(m, p))
                        ex = jnp.exp(d - m)
                        sm = ex
                        for p in perms:
                            sm = sm + _take(sm, p)
                        q = ex / sm
                        if write_q:
                            qblk[b, e, :] = q
                        for h in range(h_grp):
                            qh = _take(q, head_idx[h])
                            for k in range(kpv):
                                msg[e, pl.ds(h * in_c + k * 16, 16)] = (
                                    qh * hrow[b, e, pl.ds(k * 16, 16)])
                    if write_q:
                        pltpu.sync_copy(qblk.at[b],
                                        q_out.at[pl.ds(base, B_EDGE)])
                else:
                    pltpu.make_async_copy(q_out.at[pl.ds(base, B_EDGE)],
                                          qblk.at[b], sems[b]).wait()
                    for e in range(B_EDGE):
                        q = qblk[b, e, :]
                        for h in range(h_grp):
                            qh = _take(q, head_idx[g * h_grp + h])
                            for k in range(kpv):
                                msg[e, pl.ds(h * in_c + k * 16, 16)] = (
                                    qh * hrow[b, e, pl.ds(k * 16, 16)])
                @pl.when(i < BLOCKS_PER_SUB)
                def _scatter():
                    pltpu.sync_copy(msg, s_sh.at[idx_d.at[b]], add=True)
                    if with_deg and g == 0:
                        pltpu.sync_copy(onesv, deg_sh.at[idx_d.at[b]],
                                        add=True)

            prefetch(jnp.int32(0), 0)
            prefetch(jnp.int32(1), 1)

            def pair_body(j, carry):
                for b in range(2):
                    i = 2 * j + b
                    compute(i, b)
                    t = i + 2
                    t = jnp.where(t < BLOCKS_PER_SUB, t, 0)
                    prefetch(t, b)
                return carry

            n_pairs = (BLOCKS_PER_SUB + 1) // 2
            lax.fori_loop(0, n_pairs, pair_body, 0)
            for b in range(2):
                for d_ in gather_descs(b):
                    d_.wait()
                if g != 0:
                    pltpu.make_async_copy(q_out.at[pl.ds(ebase, B_EDGE)],
                                          qblk.at[b], sems[b]).wait()
            plsc.subcore_barrier()
            pltpu.sync_copy(s_sh.at[pl.ds(rbase, ROWS_PER_SUB)],
                            s_out.at[core, g, pl.ds(rbase, ROWS_PER_SUB)])
            if with_deg and g == 0:
                pltpu.sync_copy(deg_sh.at[pl.ds(rbase, ROWS_PER_SUB)],
                                deg_out.at[core, pl.ds(rbase, ROWS_PER_SUB)])

    return edge_kernel


def _elu(v):
    return jnp.where(v > 0, v, jnp.exp(v) - 1.0)


_BM = 1024


def _agg_call(s_all, wt, bias, deg, u):
    parts, groups = s_all.shape[0], s_all.shape[1]
    oc = wt.shape[2]
    has_deg = deg is not None
    emit_a = u is not None

    def body(*refs):
        refs = list(refs)
        s_ref = refs.pop(0)
        wt_ref = refs.pop(0)
        b_ref = refs.pop(0)
        deg_ref = refs.pop(0) if has_deg else None
        u_ref = refs.pop(0) if emit_a else None
        h_ref = refs.pop(0)
        a_ref = refs.pop(0) if emit_a else None

        acc = jnp.zeros((_BM, oc), jnp.float32)
        for p in range(parts):
            for g in range(groups):
                acc = acc + jnp.dot(s_ref[p, g], wt_ref[g],
                                    preferred_element_type=jnp.float32,
                                    precision=lax.Precision.HIGHEST)
        if has_deg:
            dg = deg_ref[0, :, 0:1] + deg_ref[1, :, 0:1]
            acc = acc / jnp.clip(dg, 1.0, None)
        h = _elu(acc + b_ref[0])
        h_ref[...] = h
        if emit_a:
            a = jnp.dot(h, u_ref[...], preferred_element_type=jnp.float32,
                        precision=lax.Precision.HIGHEST)
            a_ref[...] = jnp.concatenate([a, a], axis=1)

    grid = N_PAD // _BM
    in_specs = [
        pl.BlockSpec((parts, groups, _BM, 128), lambda i: (0, 0, i, 0)),
        pl.BlockSpec((groups, 128, oc), lambda i: (0, 0, 0)),
        pl.BlockSpec((1, oc), lambda i: (0, 0)),
    ]
    args = [s_all, wt, bias.reshape(1, oc)]
    if has_deg:
        in_specs.append(pl.BlockSpec((2, _BM, 16), lambda i: (0, i, 0)))
        args.append(deg)
    if emit_a:
        in_specs.append(pl.BlockSpec((oc, HEADS), lambda i: (0, 0)))
        args.append(u)
    out_shape = [jax.ShapeDtypeStruct((N_PAD, oc), jnp.float32)]
    out_specs = [pl.BlockSpec((_BM, oc), lambda i: (i, 0))]
    if emit_a:
        out_shape.append(jax.ShapeDtypeStruct((N_PAD, 16), jnp.float32))
        out_specs.append(pl.BlockSpec((_BM, 16), lambda i: (i, 0)))
    res = pl.pallas_call(body, grid=(grid,), in_specs=in_specs,
                         out_specs=out_specs, out_shape=out_shape)(*args)
    return res if emit_a else (res[0], None)


def _final_call(h, w1, b1, w2p, b2p):
    def body(h_ref, w1_ref, b1_ref, w2_ref, b2_ref, o_ref):
        m = _elu(jnp.dot(h_ref[...], w1_ref[...],
                         preferred_element_type=jnp.float32,
                         precision=lax.Precision.HIGHEST) + b1_ref[0])
        o_ref[...] = jnp.dot(m, w2_ref[...],
                             preferred_element_type=jnp.float32,
                             precision=lax.Precision.HIGHEST) + b2_ref[0]

    grid = N_PAD // _BM
    return pl.pallas_call(
        body, grid=(grid,),
        in_specs=[
            pl.BlockSpec((_BM, 128), lambda i: (i, 0)),
            pl.BlockSpec((128, 256), lambda i: (0, 0)),
            pl.BlockSpec((1, 256), lambda i: (0, 0)),
            pl.BlockSpec((256, 16), lambda i: (0, 0)),
            pl.BlockSpec((1, 16), lambda i: (0, 0)),
        ],
        out_specs=pl.BlockSpec((_BM, 16), lambda i: (i, 0)),
        out_shape=jax.ShapeDtypeStruct((N_PAD, 16), jnp.float32),
    )(h, w1, b1.reshape(1, 256), w2p, b2p)


def _wt_layout(lin, in_c, oc):
    w = lin.reshape(in_c, HEADS, oc).transpose(1, 0, 2).reshape(HEADS * in_c, oc)
    return w.reshape(-1, 128, oc)


_EDGE_CACHE = {}


def _edge(in_c, num_groups, with_deg):
    key = (in_c, num_groups, with_deg)
    if key not in _EDGE_CACHE:
        _EDGE_CACHE[key] = _make_edge_kernel(in_c, num_groups, with_deg)
    return _EDGE_CACHE[key]


def kernel(x, edge_index, W0, b0, lin1, u1, c1, bias1, lin2, u2, c2, bias2,
           lin3, u3, c3, bias3, W1, bfc1, W2, bfc2):
    src = edge_index[0]
    dst = edge_index[1]
    xp = jnp.pad(x, ((0, N_PAD - N_NODES), (0, 0)))
    zeros = jnp.zeros((N_PAD, 128), jnp.float32)
    zeros16 = jnp.zeros((N_PAD, 16), jnp.float32)
    ones = jnp.ones((B_EDGE, 16), jnp.float32)
    c1t = jnp.concatenate([c1, c1])
    c2t = jnp.concatenate([c2, c2])
    c3t = jnp.concatenate([c3, c3])

    h0, a1 = _agg_call(xp.reshape(1, 1, N_PAD, 128), W0.reshape(1, 128, 16),
                       b0, None, u1)
    s1, deg = _edge(16, 1, True)(src, dst, h0, a1, c1t, zeros, zeros16, ones)
    h1, a2 = _agg_call(s1, _wt_layout(lin1, 16, 32), bias1, deg, u2)
    s2, _ = _edge(32, 2, False)(src, dst, h1, a2, c2t, zeros, zeros16, ones)
    h2, a3 = _agg_call(s2, _wt_layout(lin2, 32, 64), bias2, deg, u3)
    s3, _ = _edge(64, 4, False)(src, dst, h2, a3, c3t, zeros, zeros16, ones)
    h3, _ = _agg_call(s3, _wt_layout(lin3, 64, 128), bias3, deg, None)

    w2p = jnp.zeros((256, 16), jnp.float32).at[:, :10].set(W2)
    b2p = jnp.zeros((1, 16), jnp.float32).at[0, :10].set(bfc2)
    out = _final_call(h3, W1, bfc1, w2p, b2p)
    return out[:N_NODES, :10]
